# baseline (speedup 1.0000x reference)
import dataclasses

import jax
import jax.numpy as jnp
from jax import lax
from jax.experimental import pallas as pl
from jax.experimental.pallas import tpu as pltpu
from jax.experimental.pallas import tpu_sc as plsc

B = 16384
T = 20
D = 64
V = 1048576
L = 16
NC = 2
NS = 16
NW = NC * NS
B_PER_W = B // NW
NB = 32
NCHUNK = B_PER_W // NB
ROWS = NB * T
IDX_W = 128
N_IDX_ROWS = ROWS // IDX_W
NBLK = V // 128
BLK_PER_W = NBLK // NW


def _convert_kernel(tin_hbm, tout_hbm, oin_hbm, oout_hbm,
                    blk0, blk1, rows0, rows1,
                    sin0, sin1, sout0, sout1):
    wid = lax.axis_index("s") * NC + lax.axis_index("c")
    j_lo = wid * BLK_PER_W
    j_hi = j_lo + BLK_PER_W

    lane = lax.broadcasted_iota(jnp.int32, (L,), 0)
    idx_hi = lane // 8
    idx_di = (lane % 8) * 128

    blks = (blk0, blk1)
    rows = (rows0, rows1)
    sins = (sin0, sin1)
    souts = (sout0, sout1)

    for t_hbm, o_hbm in ((tin_hbm, oin_hbm), (tout_hbm, oout_hbm)):
        for b in range(2):
            pltpu.async_copy(t_hbm.at[:, j_lo + b, :], blks[b], sins[b])

        @pl.loop(j_lo, j_hi, step=2)
        def _blk(jj):
            for b in range(2):
                j = jj + b
                pltpu.make_async_copy(
                    t_hbm.at[:, j, :], blks[b], sins[b]).wait()
                @pl.when(jj > j_lo)
                def _():
                    pltpu.make_async_copy(
                        rows[b], o_hbm.at[pl.ds(j * 128, 128), :],
                        souts[b]).wait()

                @pl.loop(0, 128, step=8)
                def _word(w0):
                    for dw in range(8):
                        wl = w0 + dw
                        col = idx_di + wl
                        g = [plsc.load_gather(blks[b], [idx_hi + 2 * k, col])
                             for k in range(4)]
                        rows[b][wl, pl.ds(0, 2 * L)] = plsc.pack(
                            g[0], g[1], format=plsc.PackFormat.INTERLEAVED)
                        rows[b][wl, pl.ds(2 * L, 2 * L)] = plsc.pack(
                            g[2], g[3], format=plsc.PackFormat.INTERLEAVED)

                pltpu.async_copy(
                    rows[b], o_hbm.at[pl.ds(j * 128, 128), :], souts[b])

                @pl.when(j + 2 < j_hi)
                def _():
                    pltpu.async_copy(t_hbm.at[:, j + 2, :], blks[b], sins[b])

        for b in range(2):
            pltpu.make_async_copy(
                rows[b], o_hbm.at[pl.ds(0, 128), :], souts[b]).wait()


def _gather_dot_kernel(cw_hbm, tw_hbm, ein_hbm, eout_hbm, out_hbm,
                       cen_idx_v, tgt_idx_v, cen_rows_v, tgt_rows_v, out_v,
                       sem):
    wid = lax.axis_index("s") * NC + lax.axis_index("c")

    @pl.loop(0, NCHUNK)
    def _chunk(ci):
        base_b = wid * B_PER_W + ci * NB

        pltpu.sync_copy(cw_hbm.at[pl.ds(base_b, NB)], cen_idx_v)
        for k in range(N_IDX_ROWS):
            pltpu.sync_copy(tw_hbm.at[pl.ds(base_b * T + k * IDX_W, IDX_W)],
                            tgt_idx_v.at[k])

        cps = [pltpu.async_copy(ein_hbm.at[cen_idx_v], cen_rows_v, sem)]
        for k in range(N_IDX_ROWS):
            cps.append(pltpu.async_copy(
                eout_hbm.at[tgt_idx_v.at[k]],
                tgt_rows_v.at[pl.ds(k * IDX_W, IDX_W)], sem))
        for cp in cps:
            cp.wait()

        lane = lax.broadcasted_iota(jnp.int32, (L,), 0)

        def row_f32(ref, r):
            h0 = plsc.unpack(ref[r, pl.ds(0, 2 * L)],
                             format=plsc.PackFormat.INTERLEAVED)
            h1 = plsc.unpack(ref[r, pl.ds(2 * L, 2 * L)],
                             format=plsc.PackFormat.INTERLEAVED)
            return (h0[0], h0[1], h1[0], h1[1])

        @pl.loop(0, NB // 4)
        def _quad(b4):
            bl = b4 * 4
            cen = [row_f32(cen_rows_v, bl + i) for i in range(4)]
            rbase = bl * T
            for g in range(5):
                w = jnp.zeros((L,), jnp.float32)
                for s in range(L):
                    j = g * L + s
                    i, t = j // T, j % T
                    r = rbase + i * T + t
                    tg = row_f32(tgt_rows_v, r)
                    acc = tg[0] * cen[i][0]
                    acc += tg[1] * cen[i][1]
                    acc += tg[2] * cen[i][2]
                    acc += tg[3] * cen[i][3]
                    w = jnp.where(lane == s, jnp.sum(acc), w)
                out_v[pl.ds(rbase + g * L, L)] = w

        pltpu.sync_copy(out_v, out_hbm.at[pl.ds(base_b * T, ROWS)])


def _native_view(table):
    return (table.T.reshape(8, 8, NBLK, 128)
            .transpose(0, 2, 1, 3).reshape(8, NBLK, 1024))


def kernel(center_word, target_word, embedding_in, embedding_out):
    cw = center_word.reshape(B)
    tw = target_word.reshape(B * T)

    cp = pltpu.CompilerParams()
    for fld, val in (("needs_layout_passes", False),
                     ("use_tc_tiling_on_sc", False)):
        if fld in pltpu.CompilerParams.__dataclass_fields__:
            cp = dataclasses.replace(cp, **{fld: val})
    mesh = plsc.VectorSubcoreMesh(core_axis_name="c", subcore_axis_name="s")

    convert = pl.kernel(
        _convert_kernel,
        out_type=(jax.ShapeDtypeStruct((V, D), jnp.bfloat16),
                  jax.ShapeDtypeStruct((V, D), jnp.bfloat16)),
        mesh=mesh,
        scratch_types=[
            pltpu.VMEM((8, 1024), jnp.float32),
            pltpu.VMEM((8, 1024), jnp.float32),
            pltpu.VMEM((128, D), jnp.bfloat16),
            pltpu.VMEM((128, D), jnp.bfloat16),
            pltpu.SemaphoreType.DMA,
            pltpu.SemaphoreType.DMA,
            pltpu.SemaphoreType.DMA,
            pltpu.SemaphoreType.DMA,
        ],
        compiler_params=cp,
    )
    ein_bf, eout_bf = convert(_native_view(embedding_in),
                              _native_view(embedding_out))

    run = pl.kernel(
        _gather_dot_kernel,
        out_type=jax.ShapeDtypeStruct((B * T,), jnp.float32),
        mesh=mesh,
        scratch_types=[
            pltpu.VMEM((NB,), jnp.int32),
            pltpu.VMEM((N_IDX_ROWS, IDX_W), jnp.int32),
            pltpu.VMEM((NB, D), jnp.bfloat16),
            pltpu.VMEM((ROWS, D), jnp.bfloat16),
            pltpu.VMEM((ROWS,), jnp.float32),
            pltpu.SemaphoreType.DMA,
        ],
        compiler_params=cp,
    )
    flat = run(cw, tw, ein_bf, eout_bf)
    return flat.reshape(B, T)


# speedup vs baseline: 1.2347x; 1.2347x over previous
import dataclasses

import jax
import jax.numpy as jnp
from jax import lax
from jax.experimental import pallas as pl
from jax.experimental.pallas import tpu as pltpu
from jax.experimental.pallas import tpu_sc as plsc

B = 16384
T = 20
D = 64
V = 1048576
L = 16
NC = 2
NS = 16
NW = NC * NS
B_PER_W = B // NW
NB = 32
NCHUNK = B_PER_W // NB
ROWS = NB * T
IDX_W = 128
N_IDX_ROWS = ROWS // IDX_W
NBLK = V // 128
BLK_PER_W = NBLK // NW


def _convert_kernel(tin_hbm, tout_hbm, oin_hbm, oout_hbm,
                    blk0, blk1, rows0, rows1,
                    sin0, sin1, sout0, sout1):
    wid = lax.axis_index("s") * NC + lax.axis_index("c")
    j_lo = wid * BLK_PER_W
    j_hi = j_lo + BLK_PER_W

    lane = lax.broadcasted_iota(jnp.int32, (L,), 0)
    idx_hi = [lane // 8 + 2 * k for k in range(4)]
    idx_di = (lane % 8) * 128

    blks = (blk0, blk1)
    rows = (rows0, rows1)
    sins = (sin0, sin1)
    souts = (sout0, sout1)

    for t_hbm, o_hbm in ((tin_hbm, oin_hbm), (tout_hbm, oout_hbm)):
        for b in range(2):
            pltpu.async_copy(t_hbm.at[:, j_lo + b, :], blks[b], sins[b])

        @pl.loop(j_lo, j_hi, step=2)
        def _blk(jj):
            for b in range(2):
                j = jj + b
                pltpu.make_async_copy(
                    t_hbm.at[:, j, :], blks[b], sins[b]).wait()
                @pl.when(jj > j_lo)
                def _():
                    pltpu.make_async_copy(
                        rows[b], o_hbm.at[pl.ds(j * 128, 128), :],
                        souts[b]).wait()

                @plsc.parallel_loop(0, 128, step=1, unroll=8)
                def _word(wl):
                    col = idx_di + wl
                    g = [plsc.load_gather(blks[b], [idx_hi[k], col])
                         for k in range(4)]
                    rows[b][wl, pl.ds(0, 2 * L)] = plsc.pack(
                        g[0], g[1], format=plsc.PackFormat.INTERLEAVED)
                    rows[b][wl, pl.ds(2 * L, 2 * L)] = plsc.pack(
                        g[2], g[3], format=plsc.PackFormat.INTERLEAVED)

                pltpu.async_copy(
                    rows[b], o_hbm.at[pl.ds(j * 128, 128), :], souts[b])

                @pl.when(j + 2 < j_hi)
                def _():
                    pltpu.async_copy(t_hbm.at[:, j + 2, :], blks[b], sins[b])

        for b in range(2):
            pltpu.make_async_copy(
                rows[b], o_hbm.at[pl.ds(0, 128), :], souts[b]).wait()


def _gather_dot_kernel(cw_hbm, tw_hbm, ein_hbm, eout_hbm, out_hbm,
                       cen_idx_v, tgt_idx_v, cen_rows_v, tgt_rows_v, out_v,
                       sem):
    wid = lax.axis_index("s") * NC + lax.axis_index("c")

    @pl.loop(0, NCHUNK)
    def _chunk(ci):
        base_b = wid * B_PER_W + ci * NB

        pltpu.sync_copy(cw_hbm.at[pl.ds(base_b, NB)], cen_idx_v)
        for k in range(N_IDX_ROWS):
            pltpu.sync_copy(tw_hbm.at[pl.ds(base_b * T + k * IDX_W, IDX_W)],
                            tgt_idx_v.at[k])

        cps = [pltpu.async_copy(ein_hbm.at[cen_idx_v], cen_rows_v, sem)]
        for k in range(N_IDX_ROWS):
            cps.append(pltpu.async_copy(
                eout_hbm.at[tgt_idx_v.at[k]],
                tgt_rows_v.at[pl.ds(k * IDX_W, IDX_W)], sem))
        for cp in cps:
            cp.wait()

        lane = lax.broadcasted_iota(jnp.int32, (L,), 0)

        def row_f32(ref, r):
            h0 = plsc.unpack(ref[r, pl.ds(0, 2 * L)],
                             format=plsc.PackFormat.INTERLEAVED)
            h1 = plsc.unpack(ref[r, pl.ds(2 * L, 2 * L)],
                             format=plsc.PackFormat.INTERLEAVED)
            return (h0[0], h0[1], h1[0], h1[1])

        @pl.loop(0, NB // 4)
        def _quad(b4):
            bl = b4 * 4
            cen = [row_f32(cen_rows_v, bl + i) for i in range(4)]
            rbase = bl * T
            for g in range(5):
                w = jnp.zeros((L,), jnp.float32)
                for s in range(L):
                    j = g * L + s
                    i, t = j // T, j % T
                    r = rbase + i * T + t
                    tg = row_f32(tgt_rows_v, r)
                    acc = tg[0] * cen[i][0]
                    acc += tg[1] * cen[i][1]
                    acc += tg[2] * cen[i][2]
                    acc += tg[3] * cen[i][3]
                    w = jnp.where(lane == s, jnp.sum(acc), w)
                out_v[pl.ds(rbase + g * L, L)] = w

        pltpu.sync_copy(out_v, out_hbm.at[pl.ds(base_b * T, ROWS)])


def _native_view(table):
    return (table.T.reshape(8, 8, NBLK, 128)
            .transpose(0, 2, 1, 3).reshape(8, NBLK, 1024))


def kernel(center_word, target_word, embedding_in, embedding_out):
    cw = center_word.reshape(B)
    tw = target_word.reshape(B * T)

    cp = pltpu.CompilerParams()
    for fld, val in (("needs_layout_passes", False),
                     ("use_tc_tiling_on_sc", False)):
        if fld in pltpu.CompilerParams.__dataclass_fields__:
            cp = dataclasses.replace(cp, **{fld: val})
    mesh = plsc.VectorSubcoreMesh(core_axis_name="c", subcore_axis_name="s")

    convert = pl.kernel(
        _convert_kernel,
        out_type=(jax.ShapeDtypeStruct((V, D), jnp.bfloat16),
                  jax.ShapeDtypeStruct((V, D), jnp.bfloat16)),
        mesh=mesh,
        scratch_types=[
            pltpu.VMEM((8, 1024), jnp.float32),
            pltpu.VMEM((8, 1024), jnp.float32),
            pltpu.VMEM((128, D), jnp.bfloat16),
            pltpu.VMEM((128, D), jnp.bfloat16),
            pltpu.SemaphoreType.DMA,
            pltpu.SemaphoreType.DMA,
            pltpu.SemaphoreType.DMA,
            pltpu.SemaphoreType.DMA,
        ],
        compiler_params=cp,
    )
    ein_bf, eout_bf = convert(_native_view(embedding_in),
                              _native_view(embedding_out))

    run = pl.kernel(
        _gather_dot_kernel,
        out_type=jax.ShapeDtypeStruct((B * T,), jnp.float32),
        mesh=mesh,
        scratch_types=[
            pltpu.VMEM((NB,), jnp.int32),
            pltpu.VMEM((N_IDX_ROWS, IDX_W), jnp.int32),
            pltpu.VMEM((NB, D), jnp.bfloat16),
            pltpu.VMEM((ROWS, D), jnp.bfloat16),
            pltpu.VMEM((ROWS,), jnp.float32),
            pltpu.SemaphoreType.DMA,
        ],
        compiler_params=cp,
    )
    flat = run(cw, tw, ein_bf, eout_bf)
    return flat.reshape(B, T)


# speedup vs baseline: 3.8243x; 3.0974x over previous
import dataclasses

import jax
import jax.numpy as jnp
from jax import lax
from jax.experimental import pallas as pl
from jax.experimental.pallas import tpu as pltpu
from jax.experimental.pallas import tpu_sc as plsc

B = 16384
T = 20
D = 64
V = 1048576
L = 16
NC = 2
NS = 16
NW = NC * NS
B_PER_W = B // NW
NB = 32
NCHUNK = B_PER_W // NB
ROWS = NB * T
IDX_W = 128
N_IDX_ROWS = ROWS // IDX_W
NBLK = V // 128
BLK_PER_W = NBLK // NW


def _convert_kernel(tin_hbm, tout_hbm, oin_hbm, oout_hbm,
                    blk0, blk1, rows0, rows1,
                    sin0, sin1, sout0, sout1):
    wid = lax.axis_index("s") * NC + lax.axis_index("c")
    j_lo = wid * BLK_PER_W
    j_hi = j_lo + BLK_PER_W

    lane = lax.broadcasted_iota(jnp.int32, (L,), 0)
    idx_w = [lane + 16 * g for g in range(8)]

    blks = (blk0, blk1)
    rows = (rows0, rows1)
    sins = (sin0, sin1)
    souts = (sout0, sout1)

    for t_hbm, o_hbm in ((tin_hbm, oin_hbm), (tout_hbm, oout_hbm)):
        for b in range(2):
            pltpu.async_copy(t_hbm.at[:, j_lo + b, :], blks[b], sins[b])

        @pl.loop(j_lo, j_hi, step=2)
        def _blk(jj):
            for b in range(2):
                j = jj + b
                pltpu.make_async_copy(
                    t_hbm.at[:, j, :], blks[b], sins[b]).wait()
                @pl.when(jj > j_lo)
                def _():
                    pltpu.make_async_copy(
                        rows[b].at[:, pl.ds(0, D)],
                        o_hbm.at[pl.ds(j * 128, 128), :],
                        souts[b]).wait()

                @plsc.parallel_loop(0, D, step=1, unroll=4)
                def _dim(d):
                    i = d // 8
                    base = (d % 8) * 128
                    col = jnp.full((L,), d, jnp.int32)
                    for g in range(8):
                        v = blks[b][i, pl.ds(base + 16 * g, L)]
                        plsc.store_scatter(rows[b], [idx_w[g], col], v)

                pltpu.async_copy(
                    rows[b].at[:, pl.ds(0, D)],
                    o_hbm.at[pl.ds(j * 128, 128), :], souts[b])

                @pl.when(j + 2 < j_hi)
                def _():
                    pltpu.async_copy(t_hbm.at[:, j + 2, :], blks[b], sins[b])

        for b in range(2):
            pltpu.make_async_copy(
                rows[b].at[:, pl.ds(0, D)],
                o_hbm.at[pl.ds(0, 128), :], souts[b]).wait()


def _gather_dot_kernel(cw_hbm, tw_hbm, ein_hbm, eout_hbm, out_hbm,
                       cen_idx_v, tgt_idx_v, cen_rows_v, tgt_rows_v, out_v,
                       sem):
    wid = lax.axis_index("s") * NC + lax.axis_index("c")

    @pl.loop(0, NCHUNK)
    def _chunk(ci):
        base_b = wid * B_PER_W + ci * NB

        pltpu.sync_copy(cw_hbm.at[pl.ds(base_b, NB)], cen_idx_v)
        for k in range(N_IDX_ROWS):
            pltpu.sync_copy(tw_hbm.at[pl.ds(base_b * T + k * IDX_W, IDX_W)],
                            tgt_idx_v.at[k])

        cps = [pltpu.async_copy(ein_hbm.at[cen_idx_v], cen_rows_v, sem)]
        for k in range(N_IDX_ROWS):
            cps.append(pltpu.async_copy(
                eout_hbm.at[tgt_idx_v.at[k]],
                tgt_rows_v.at[pl.ds(k * IDX_W, IDX_W)], sem))
        for cp in cps:
            cp.wait()

        lane = lax.broadcasted_iota(jnp.int32, (L,), 0)

        def row_f32(ref, r):
            return tuple(ref[r, pl.ds(k * L, L)] for k in range(4))

        @pl.loop(0, NB // 4)
        def _quad(b4):
            bl = b4 * 4
            cen = [row_f32(cen_rows_v, bl + i) for i in range(4)]
            rbase = bl * T
            for g in range(5):
                w = jnp.zeros((L,), jnp.float32)
                for s in range(L):
                    j = g * L + s
                    i, t = j // T, j % T
                    r = rbase + i * T + t
                    tg = row_f32(tgt_rows_v, r)
                    acc = tg[0] * cen[i][0]
                    acc += tg[1] * cen[i][1]
                    acc += tg[2] * cen[i][2]
                    acc += tg[3] * cen[i][3]
                    w = jnp.where(lane == s, jnp.sum(acc), w)
                out_v[pl.ds(rbase + g * L, L)] = w

        pltpu.sync_copy(out_v, out_hbm.at[pl.ds(base_b * T, ROWS)])


def _native_view(table):
    return (table.T.reshape(8, 8, NBLK, 128)
            .transpose(0, 2, 1, 3).reshape(8, NBLK, 1024))


def kernel(center_word, target_word, embedding_in, embedding_out):
    cw = center_word.reshape(B)
    tw = target_word.reshape(B * T)

    cp = pltpu.CompilerParams()
    for fld, val in (("needs_layout_passes", False),
                     ("use_tc_tiling_on_sc", False)):
        if fld in pltpu.CompilerParams.__dataclass_fields__:
            cp = dataclasses.replace(cp, **{fld: val})
    mesh = plsc.VectorSubcoreMesh(core_axis_name="c", subcore_axis_name="s")

    convert = pl.kernel(
        _convert_kernel,
        out_type=(jax.ShapeDtypeStruct((V, D), jnp.float32),
                  jax.ShapeDtypeStruct((V, D), jnp.float32)),
        mesh=mesh,
        scratch_types=[
            pltpu.VMEM((8, 1024), jnp.float32),
            pltpu.VMEM((8, 1024), jnp.float32),
            pltpu.VMEM((128, D + 1), jnp.float32),
            pltpu.VMEM((128, D + 1), jnp.float32),
            pltpu.SemaphoreType.DMA,
            pltpu.SemaphoreType.DMA,
            pltpu.SemaphoreType.DMA,
            pltpu.SemaphoreType.DMA,
        ],
        compiler_params=cp,
    )
    ein_bf, eout_bf = convert(_native_view(embedding_in),
                              _native_view(embedding_out))

    run = pl.kernel(
        _gather_dot_kernel,
        out_type=jax.ShapeDtypeStruct((B * T,), jnp.float32),
        mesh=mesh,
        scratch_types=[
            pltpu.VMEM((NB,), jnp.int32),
            pltpu.VMEM((N_IDX_ROWS, IDX_W), jnp.int32),
            pltpu.VMEM((NB, D), jnp.float32),
            pltpu.VMEM((ROWS, D), jnp.float32),
            pltpu.VMEM((ROWS,), jnp.float32),
            pltpu.SemaphoreType.DMA,
        ],
        compiler_params=cp,
    )
    flat = run(cw, tw, ein_bf, eout_bf)
    return flat.reshape(B, T)
